# baseline (speedup 1.0000x reference)
import functools

import jax
import jax.numpy as jnp
from jax import lax
from jax.experimental import pallas as pl
from jax.experimental.pallas import tpu as pltpu
from jax.experimental.pallas import tpu_sc as plsc

_N = 16384
_D = 16
_K_KDE = 100
_K_RIPS = 15
_SCALE = 0.5

_R = 128
_NITER = 18
_CAP = 128
_RIPS_CAND = 48
_BUF = 160

_SC_WORKERS = 32
_ROWS_PER_W = _N // _SC_WORKERS
_UNROLL = 4
_LANES = 16


def _dist_thresh_body(q_ref, dbt_ref, qsq_ref, dbsq_ref, d_ref, thr_ref):
    qb = q_ref[...]
    dbt = dbt_ref[...]
    mm = lax.dot_general(qb, dbt, (((1,), (0,)), ((), ())),
                         preferred_element_type=jnp.float32)
    qsq = qsq_ref[0, 0, :].reshape(_R, 1)
    dbsq = dbsq_ref[0, 0, :].reshape(1, _N)
    d = (qsq + dbsq) - 2.0 * mm
    d_ref[...] = d

    lo = jnp.min(d, axis=1, keepdims=True)
    hi = jnp.max(d, axis=1, keepdims=True)

    def step(_, carry):
        lo, hi = carry
        mid = 0.5 * (lo + hi)
        cnt = jnp.sum((d <= mid).astype(jnp.float32), axis=1, keepdims=True)
        ge = cnt >= float(_K_KDE)
        hi = jnp.where(ge, mid, hi)
        lo = jnp.where(ge, lo, mid)
        return lo, hi

    lo, hi = lax.fori_loop(0, _NITER, step, (lo, hi))
    thr_ref[...] = jnp.broadcast_to(hi, (_R, _LANES))


def _dist_and_thresholds(x, qsq):
    dbt = x.T
    qsq3 = qsq.reshape(_N // _R, 1, _R)
    dbsq3 = qsq.reshape(1, 1, _N)
    d, thr = pl.pallas_call(
        _dist_thresh_body,
        grid=(_N // _R,),
        in_specs=[
            pl.BlockSpec((_R, _D), lambda i: (i, 0)),
            pl.BlockSpec((_D, _N), lambda i: (0, 0)),
            pl.BlockSpec((1, 1, _R), lambda i: (i, 0, 0)),
            pl.BlockSpec((1, 1, _N), lambda i: (0, 0, 0)),
        ],
        out_specs=[
            pl.BlockSpec((_R, _N), lambda i: (i, 0)),
            pl.BlockSpec((_R, _LANES), lambda i: (i, 0)),
        ],
        out_shape=[
            jax.ShapeDtypeStruct((_N, _N), jnp.float32),
            jax.ShapeDtypeStruct((_N, _LANES), jnp.float32),
        ],
    )(x, dbt, qsq3, dbsq3)
    return d, thr


def _sc_compact_kernel(d_hbm, thr_hbm, dc_hbm, ic_hbm,
                       rbufA, rbufB, thrv, dbuf, ibuf, curref,
                       semA, semB):
    wid = lax.axis_index("s") * 2 + lax.axis_index("c")
    row0 = wid * _ROWS_PER_W

    pltpu.sync_copy(thr_hbm.at[pl.ds(row0 * _LANES, _ROWS_PER_W * _LANES)],
                    thrv)

    def _issue(row, buf, sem):
        r = jnp.minimum(row, _N - 1)
        pltpu.make_async_copy(d_hbm.at[pl.ds(r * _N, _N)], buf, sem).start()

    def _wait(buf, sem):
        pltpu.make_async_copy(d_hbm.at[pl.ds(0, _N)], buf, sem).wait()

    _issue(row0, rbufA, semA)

    def _process(rl, buf):
        row = row0 + rl
        tv = thrv[pl.ds(rl * _LANES, _LANES)]

        for k in range(_BUF // _LANES):
            dbuf[pl.ds(k * _LANES, _LANES)] = jnp.full(
                (_LANES,), jnp.inf, dtype=jnp.float32)
            ibuf[pl.ds(k * _LANES, _LANES)] = jnp.zeros((_LANES,), jnp.int32)
        curref[...] = jnp.zeros((_LANES,), jnp.int32)

        def vec_body(g, c2):
            base = g * (_UNROLL * _LANES)
            xs, ms = [], []
            anym = None
            for u in range(_UNROLL):
                xv = buf[pl.ds(base + u * _LANES, _LANES)]
                m = xv <= tv
                xs.append(xv)
                ms.append(m)
                anym = m if anym is None else (anym | m)
            tot = jnp.sum(anym.astype(jnp.int32))

            @pl.when(tot > 0)
            def _():
                cur = curref[...]
                one16 = jnp.full((_LANES,), 1, dtype=jnp.int32)
                for u in range(_UNROLL):
                    pos = (cur + plsc.cumsum(ms[u].astype(jnp.int32))) - one16
                    civ = (jnp.full((_LANES,), base + u * _LANES, jnp.int32)
                           + lax.iota(jnp.int32, _LANES))
                    plsc.store_scatter(dbuf, [pos], xs[u], mask=ms[u])
                    plsc.store_scatter(ibuf, [pos], civ, mask=ms[u])
                    cur = cur + plsc.all_reduce_population_count(ms[u])
                curref[...] = cur

            return c2

        lax.fori_loop(0, _N // (_UNROLL * _LANES), vec_body, jnp.int32(0))

        def _ce(a, b):
            (ka, va), (kb, vb) = a, b
            sel = ka <= kb
            lo = (jnp.where(sel, ka, kb), jnp.where(sel, va, vb))
            hi = (jnp.where(sel, kb, ka), jnp.where(sel, vb, va))
            return lo, hi

        def _vsort(kv):
            sk, sv = plsc.sort_key_val(kv[0], kv[1])
            return (sk, sv)

        def _cleanup(xs):
            if len(xs) == 1:
                return [_vsort(xs[0])]
            half = len(xs) // 2
            los, his = [], []
            for i in range(half):
                lo, hi = _ce(xs[i], xs[i + half])
                los.append(lo)
                his.append(hi)
            return _cleanup(los) + _cleanup(his)

        def _bmerge(A, B):
            los, his = [], []
            for i in range(len(A)):
                kb, vb = B[len(A) - 1 - i]
                lo, hi = _ce(A[i], (lax.rev(kb, (0,)), lax.rev(vb, (0,))))
                los.append(lo)
                his.append(hi)
            return _cleanup(los) + _cleanup(his)

        runs = []
        for k in range(_CAP // _LANES):
            runs.append([_vsort((dbuf[pl.ds(k * _LANES, _LANES)],
                                 ibuf[pl.ds(k * _LANES, _LANES)]))])
        while len(runs) > 1:
            runs = [_bmerge(runs[i], runs[i + 1])
                    for i in range(0, len(runs), 2)]
        for k, (kk, vv) in enumerate(runs[0]):
            dbuf[pl.ds(k * _LANES, _LANES)] = kk
            ibuf[pl.ds(k * _LANES, _LANES)] = vv

        pltpu.sync_copy(dbuf.at[pl.ds(0, _CAP)],
                        dc_hbm.at[pl.ds(row * _CAP, _CAP)])
        pltpu.sync_copy(ibuf.at[pl.ds(0, _CAP)],
                        ic_hbm.at[pl.ds(row * _CAP, _CAP)])

    def pair_body(p, _):
        rl = p * 2
        _wait(rbufA, semA)
        _issue(row0 + rl + 1, rbufB, semB)
        _process(rl, rbufA)
        _wait(rbufB, semB)
        _issue(row0 + rl + 2, rbufA, semA)
        _process(rl + 1, rbufB)
        return _

    lax.fori_loop(0, _ROWS_PER_W // 2, pair_body, jnp.int32(0))
    _wait(rbufA, semA)


def _sc_compact(d_flat, thr_flat):
    mesh = plsc.VectorSubcoreMesh(core_axis_name="c", subcore_axis_name="s")
    f = pl.kernel(
        _sc_compact_kernel,
        mesh=mesh,
        compiler_params=pltpu.CompilerParams(needs_layout_passes=False),
        out_type=[
            jax.ShapeDtypeStruct((_N * _CAP,), jnp.float32),
            jax.ShapeDtypeStruct((_N * _CAP,), jnp.int32),
        ],
        scratch_types=[
            pltpu.VMEM((_N,), jnp.float32),
            pltpu.VMEM((_N,), jnp.float32),
            pltpu.VMEM((_ROWS_PER_W * _LANES,), jnp.float32),
            pltpu.VMEM((_BUF,), jnp.float32),
            pltpu.VMEM((_BUF,), jnp.int32),
            pltpu.VMEM((_LANES,), jnp.int32),
            pltpu.SemaphoreType.DMA,
            pltpu.SemaphoreType.DMA,
        ],
    )
    return f(d_flat, thr_flat)


def _sc_remap_kernel(sidx_hbm, ic_hbm, out_hbm, sbuf, inv, rbuf):
    wid = lax.axis_index("s") * 2 + lax.axis_index("c")

    pltpu.sync_copy(sidx_hbm, sbuf)

    def build(k, c):
        sv = sbuf[pl.ds(k * _LANES, _LANES)]
        posv = (jnp.full((_LANES,), k * _LANES, jnp.int32)
                + lax.iota(jnp.int32, _LANES))
        plsc.store_scatter(inv, [sv], posv)
        return c

    lax.fori_loop(0, _N // _LANES, build, jnp.int32(0))

    npw = _ROWS_PER_W * _RIPS_CAND
    base = wid * npw
    pltpu.sync_copy(ic_hbm.at[pl.ds(base, npw)], rbuf)

    def remap(k, c):
        iv = rbuf[pl.ds(k * _LANES, _LANES)]
        rbuf[pl.ds(k * _LANES, _LANES)] = plsc.load_gather(inv, [iv])
        return c

    lax.fori_loop(0, npw // _LANES, remap, jnp.int32(0))
    pltpu.sync_copy(rbuf, out_hbm.at[pl.ds(base, npw)])


def _sc_remap(sorted_idxs, ic48_flat):
    mesh = plsc.VectorSubcoreMesh(core_axis_name="c", subcore_axis_name="s")
    f = pl.kernel(
        _sc_remap_kernel,
        mesh=mesh,
        compiler_params=pltpu.CompilerParams(needs_layout_passes=False),
        out_type=jax.ShapeDtypeStruct((_N * _RIPS_CAND,), jnp.int32),
        scratch_types=[
            pltpu.VMEM((_N,), jnp.int32),
            pltpu.VMEM((_N,), jnp.int32),
            pltpu.VMEM((_ROWS_PER_W * _RIPS_CAND,), jnp.int32),
        ],
    )
    return f(sorted_idxs, ic48_flat)


def kernel(x):
    x = x.astype(jnp.float32)
    qsq = jnp.sum(x * x, axis=1)

    d, thr = _dist_and_thresholds(x, qsq)

    dc_flat, ic_flat = _sc_compact(d.reshape(-1), thr.reshape(-1))
    dc = dc_flat.reshape(_N, _CAP)
    ic = ic_flat.reshape(_N, _CAP)

    ds_, is_ = dc, ic

    vals = ds_[:, :_K_KDE]
    density = jnp.sum(jnp.exp(-vals / _SCALE), axis=1) / (_K_KDE * _SCALE)
    density = density / jnp.max(density)

    sorted_idxs = jnp.argsort(density)
    density_sorted = density[sorted_idxs]

    dss = ds_[:, :_RIPS_CAND]
    ic48_flat = is_[:, :_RIPS_CAND].reshape(-1)
    pidx = _sc_remap(sorted_idxs, ic48_flat).reshape(_N, _RIPS_CAND)
    pidx_s, ds2 = lax.sort((pidx, dss), num_keys=1, is_stable=True)
    _, ps = lax.sort((ds2, pidx_s), num_keys=1, is_stable=True)
    rips_rows = ps[:, :_K_RIPS]
    rips_idxs = rips_rows[sorted_idxs]

    return density_sorted, rips_idxs


# speedup vs baseline: 1.1194x; 1.1194x over previous
import functools

import jax
import jax.numpy as jnp
from jax import lax
from jax.experimental import pallas as pl
from jax.experimental.pallas import tpu as pltpu
from jax.experimental.pallas import tpu_sc as plsc

_N = 16384
_D = 16
_K_KDE = 100
_K_RIPS = 15
_SCALE = 0.5

_R = 128
_NITER = 18
_CAP = 128
_RIPS_CAND = 48
_BUF = 160

_SC_WORKERS = 32
_ROWS_PER_W = _N // _SC_WORKERS
_ROWS_PER_H = (_N // 2) // _SC_WORKERS
_UNROLL = 4
_LANES = 16


def _dist_thresh_body(q_ref, dbt_ref, qsq_ref, dbsq_ref, d_ref, thr_ref):
    qb = q_ref[...]
    dbt = dbt_ref[...]
    mm = lax.dot_general(qb, dbt, (((1,), (0,)), ((), ())),
                         preferred_element_type=jnp.float32)
    qsq = qsq_ref[0, 0, :].reshape(_R, 1)
    dbsq = dbsq_ref[0, 0, :].reshape(1, _N)
    d = (qsq + dbsq) - 2.0 * mm
    d_ref[...] = d

    lo = jnp.min(d, axis=1, keepdims=True)
    hi = jnp.max(d, axis=1, keepdims=True)

    def step(_, carry):
        lo, hi = carry
        mid = 0.5 * (lo + hi)
        cnt = jnp.sum((d <= mid).astype(jnp.float32), axis=1, keepdims=True)
        ge = cnt >= float(_K_KDE)
        hi = jnp.where(ge, mid, hi)
        lo = jnp.where(ge, lo, mid)
        return lo, hi

    lo, hi = lax.fori_loop(0, _NITER, step, (lo, hi))
    thr_ref[...] = jnp.broadcast_to(hi, (_R, _LANES))


def _dist_and_thresholds(x, qsq, half):
    dbt = x.T
    qsq3 = qsq.reshape(_N // _R, 1, _R)
    dbsq3 = qsq.reshape(1, 1, _N)
    nh = _N // 2
    off = half * (nh // _R)
    d, thr = pl.pallas_call(
        _dist_thresh_body,
        grid=(nh // _R,),
        in_specs=[
            pl.BlockSpec((_R, _D), lambda i: (i + off, 0)),
            pl.BlockSpec((_D, _N), lambda i: (0, 0)),
            pl.BlockSpec((1, 1, _R), lambda i: (i + off, 0, 0)),
            pl.BlockSpec((1, 1, _N), lambda i: (0, 0, 0)),
        ],
        out_specs=[
            pl.BlockSpec((_R, _N), lambda i: (i, 0)),
            pl.BlockSpec((_R, _LANES), lambda i: (i, 0)),
        ],
        out_shape=[
            jax.ShapeDtypeStruct((nh, _N), jnp.float32),
            jax.ShapeDtypeStruct((nh, _LANES), jnp.float32),
        ],
    )(x, dbt, qsq3, dbsq3)
    return d, thr


def _sc_compact_kernel(d_hbm, thr_hbm, dc_hbm, ic_hbm,
                       rbufA, rbufB, thrv, dbuf, ibuf, curref,
                       semA, semB):
    wid = lax.axis_index("s") * 2 + lax.axis_index("c")
    row0 = wid * _ROWS_PER_H

    pltpu.sync_copy(thr_hbm.at[pl.ds(row0 * _LANES, _ROWS_PER_H * _LANES)],
                    thrv)

    def _issue(row, buf, sem):
        r = jnp.minimum(row, _N - 1)
        pltpu.make_async_copy(d_hbm.at[pl.ds(r * _N, _N)], buf, sem).start()

    def _wait(buf, sem):
        pltpu.make_async_copy(d_hbm.at[pl.ds(0, _N)], buf, sem).wait()

    _issue(row0, rbufA, semA)

    def _process(rl, buf):
        row = row0 + rl
        tv = thrv[pl.ds(rl * _LANES, _LANES)]

        for k in range(_BUF // _LANES):
            dbuf[pl.ds(k * _LANES, _LANES)] = jnp.full(
                (_LANES,), jnp.inf, dtype=jnp.float32)
            ibuf[pl.ds(k * _LANES, _LANES)] = jnp.zeros((_LANES,), jnp.int32)
        curref[...] = jnp.zeros((_LANES,), jnp.int32)

        def vec_body(g, c2):
            base = g * (_UNROLL * _LANES)
            xs, ms, cs = [], [], []
            tot = jnp.int32(0)
            for u in range(_UNROLL):
                xv = buf[pl.ds(base + u * _LANES, _LANES)]
                m = xv <= tv
                xs.append(xv)
                ms.append(m)
                c = jnp.sum(m.astype(jnp.int32))
                cs.append(c)
                tot = tot + c

            @pl.when(tot > 0)
            def _():
                cur = curref[...]
                one16 = jnp.full((_LANES,), 1, dtype=jnp.int32)
                for u in range(_UNROLL):
                    pos = (cur + plsc.cumsum(ms[u].astype(jnp.int32))) - one16
                    civ = (jnp.full((_LANES,), base + u * _LANES, jnp.int32)
                           + lax.iota(jnp.int32, _LANES))
                    plsc.store_scatter(dbuf, [pos], xs[u], mask=ms[u])
                    plsc.store_scatter(ibuf, [pos], civ, mask=ms[u])
                    cur = cur + plsc.all_reduce_population_count(ms[u])
                curref[...] = cur

            return c2

        lax.fori_loop(0, _N // (_UNROLL * _LANES), vec_body, jnp.int32(0))

        def _ce(a, b):
            (ka, va), (kb, vb) = a, b
            sel = ka <= kb
            lo = (jnp.where(sel, ka, kb), jnp.where(sel, va, vb))
            hi = (jnp.where(sel, kb, ka), jnp.where(sel, vb, va))
            return lo, hi

        def _vsort(kv):
            sk, sv = plsc.sort_key_val(kv[0], kv[1])
            return (sk, sv)

        def _cleanup(xs):
            if len(xs) == 1:
                return [_vsort(xs[0])]
            half = len(xs) // 2
            los, his = [], []
            for i in range(half):
                lo, hi = _ce(xs[i], xs[i + half])
                los.append(lo)
                his.append(hi)
            return _cleanup(los) + _cleanup(his)

        def _bmerge(A, B):
            los, his = [], []
            for i in range(len(A)):
                kb, vb = B[len(A) - 1 - i]
                lo, hi = _ce(A[i], (lax.rev(kb, (0,)), lax.rev(vb, (0,))))
                los.append(lo)
                his.append(hi)
            return _cleanup(los) + _cleanup(his)

        runs = []
        for k in range(_CAP // _LANES):
            runs.append([_vsort((dbuf[pl.ds(k * _LANES, _LANES)],
                                 ibuf[pl.ds(k * _LANES, _LANES)]))])
        while len(runs) > 1:
            runs = [_bmerge(runs[i], runs[i + 1])
                    for i in range(0, len(runs), 2)]
        for k, (kk, vv) in enumerate(runs[0]):
            dbuf[pl.ds(k * _LANES, _LANES)] = kk
            ibuf[pl.ds(k * _LANES, _LANES)] = vv

        pltpu.sync_copy(dbuf.at[pl.ds(0, _CAP)],
                        dc_hbm.at[pl.ds(row * _CAP, _CAP)])
        pltpu.sync_copy(ibuf.at[pl.ds(0, _CAP)],
                        ic_hbm.at[pl.ds(row * _CAP, _CAP)])

    def pair_body(p, _):
        rl = p * 2
        _wait(rbufA, semA)
        _issue(row0 + rl + 1, rbufB, semB)
        _process(rl, rbufA)
        _wait(rbufB, semB)
        _issue(row0 + rl + 2, rbufA, semA)
        _process(rl + 1, rbufB)
        return _

    lax.fori_loop(0, _ROWS_PER_H // 2, pair_body, jnp.int32(0))
    _wait(rbufA, semA)


def _sc_compact(d_flat, thr_flat):
    mesh = plsc.VectorSubcoreMesh(core_axis_name="c", subcore_axis_name="s")
    f = pl.kernel(
        _sc_compact_kernel,
        mesh=mesh,
        compiler_params=pltpu.CompilerParams(needs_layout_passes=False),
        out_type=[
            jax.ShapeDtypeStruct(((_N // 2) * _CAP,), jnp.float32),
            jax.ShapeDtypeStruct(((_N // 2) * _CAP,), jnp.int32),
        ],
        scratch_types=[
            pltpu.VMEM((_N,), jnp.float32),
            pltpu.VMEM((_N,), jnp.float32),
            pltpu.VMEM((_ROWS_PER_H * _LANES,), jnp.float32),
            pltpu.VMEM((_BUF,), jnp.float32),
            pltpu.VMEM((_BUF,), jnp.int32),
            pltpu.VMEM((_LANES,), jnp.int32),
            pltpu.SemaphoreType.DMA,
            pltpu.SemaphoreType.DMA,
        ],
    )
    return f(d_flat, thr_flat)


def _sc_remap_kernel(sidx_hbm, ic_hbm, out_hbm, sbuf, inv, rbuf):
    wid = lax.axis_index("s") * 2 + lax.axis_index("c")

    pltpu.sync_copy(sidx_hbm, sbuf)

    def build(k, c):
        sv = sbuf[pl.ds(k * _LANES, _LANES)]
        posv = (jnp.full((_LANES,), k * _LANES, jnp.int32)
                + lax.iota(jnp.int32, _LANES))
        plsc.store_scatter(inv, [sv], posv)
        return c

    lax.fori_loop(0, _N // _LANES, build, jnp.int32(0))

    npw = _ROWS_PER_W * _RIPS_CAND
    base = wid * npw
    pltpu.sync_copy(ic_hbm.at[pl.ds(base, npw)], rbuf)

    def remap(k, c):
        iv = rbuf[pl.ds(k * _LANES, _LANES)]
        rbuf[pl.ds(k * _LANES, _LANES)] = plsc.load_gather(inv, [iv])
        return c

    lax.fori_loop(0, npw // _LANES, remap, jnp.int32(0))
    pltpu.sync_copy(rbuf, out_hbm.at[pl.ds(base, npw)])


def _sc_remap(sorted_idxs, ic48_flat):
    mesh = plsc.VectorSubcoreMesh(core_axis_name="c", subcore_axis_name="s")
    f = pl.kernel(
        _sc_remap_kernel,
        mesh=mesh,
        compiler_params=pltpu.CompilerParams(needs_layout_passes=False),
        out_type=jax.ShapeDtypeStruct((_N * _RIPS_CAND,), jnp.int32),
        scratch_types=[
            pltpu.VMEM((_N,), jnp.int32),
            pltpu.VMEM((_N,), jnp.int32),
            pltpu.VMEM((_ROWS_PER_W * _RIPS_CAND,), jnp.int32),
        ],
    )
    return f(sorted_idxs, ic48_flat)


def kernel(x):
    x = x.astype(jnp.float32)
    qsq = jnp.sum(x * x, axis=1)

    dA, thrA = _dist_and_thresholds(x, qsq, 0)
    dB, thrB = _dist_and_thresholds(x, qsq, 1)
    dcA, icA = _sc_compact(dA.reshape(-1), thrA.reshape(-1))
    dcB, icB = _sc_compact(dB.reshape(-1), thrB.reshape(-1))
    dc = jnp.concatenate([dcA, dcB]).reshape(_N, _CAP)
    ic = jnp.concatenate([icA, icB]).reshape(_N, _CAP)

    ds_, is_ = dc, ic

    vals = ds_[:, :_K_KDE]
    density = jnp.sum(jnp.exp(-vals / _SCALE), axis=1) / (_K_KDE * _SCALE)
    density = density / jnp.max(density)

    sorted_idxs = jnp.argsort(density)
    density_sorted = density[sorted_idxs]

    dss = ds_[:, :_RIPS_CAND]
    ic48_flat = is_[:, :_RIPS_CAND].reshape(-1)
    pidx = _sc_remap(sorted_idxs, ic48_flat).reshape(_N, _RIPS_CAND)
    pidx_s, ds2 = lax.sort((pidx, dss), num_keys=1, is_stable=True)
    _, ps = lax.sort((ds2, pidx_s), num_keys=1, is_stable=True)
    rips_rows = ps[:, :_K_RIPS]
    rips_idxs = rips_rows[sorted_idxs]

    return density_sorted, rips_idxs
